# Initial kernel scaffold; baseline (speedup 1.0000x reference)
import jax
import jax.numpy as jnp
from jax.experimental import pallas as pl


def kernel(x, edge_index, edge_attr, edge_weights, batch, W1, b1, W2, b2, W3, b3, Wf, bf):
    raise NotImplementedError("write your pallas kernel here")


# speedup vs baseline: 12.5454x; 12.5454x over previous
import functools

import jax
import jax.numpy as jnp
from jax import lax
from jax.experimental import pallas as pl
from jax.experimental.pallas import tpu as pltpu
from jax.experimental.pallas import tpu_sc as plsc

N = 50000
E = 800000
NPAD = 51200
EPAD = 819200
ROWS_PER_SUB = 3200
EDGES_PER_TILE = EPAD // 32
CHUNK = 128
NCHUNK = EDGES_PER_TILE // CHUNK
BLK = 2048
GRID = NPAD // BLK
G = 64
F32 = jnp.float32


def _zero_acc_slice(zbuf, acc, s):
    def z(k, _):
        pltpu.sync_copy(zbuf, acc.at[pl.ds(s * ROWS_PER_SUB + k * 128, 128)])
        return _
    lax.fori_loop(0, 25, z, 0)


def _make_sc_agg(F):
    mesh = plsc.VectorSubcoreMesh(core_axis_name="c", subcore_axis_name="s")

    @functools.partial(
        pl.kernel,
        out_type=jax.ShapeDtypeStruct((2, NPAD, F), F32),
        mesh=mesh,
        compiler_params=pltpu.CompilerParams(use_tc_tiling_on_sc=False),
        scratch_types=[
            pltpu.VMEM((CHUNK,), jnp.int32),
            pltpu.VMEM((CHUNK,), jnp.int32),
            pltpu.VMEM((CHUNK,), F32),
            pltpu.VMEM((CHUNK, F), F32),
            pltpu.VMEM_SHARED((NPAD, F), F32),
            pltpu.SemaphoreType.DMA,
        ],
    )
    def sc_agg(h_hbm, src_hbm, dst_hbm, ew_hbm, out_hbm,
               srcv, dstv, ewv, rows, acc, sem):
        c = lax.axis_index("c")
        s = lax.axis_index("s")

        def zb(i, _):
            for t in range(F // 16):
                rows[i, pl.ds(t * 16, 16)] = jnp.zeros((16,), F32)
            return _
        lax.fori_loop(0, CHUNK, zb, 0)
        _zero_acc_slice(rows, acc, s)
        plsc.subcore_barrier()

        base = (c * 16 + s) * EDGES_PER_TILE

        def chunk_body(i, _):
            e0 = base + i * CHUNK
            pltpu.sync_copy(src_hbm.at[pl.ds(e0, CHUNK)], srcv)
            pltpu.sync_copy(dst_hbm.at[pl.ds(e0, CHUNK)], dstv)
            pltpu.sync_copy(ew_hbm.at[pl.ds(e0, CHUNK)], ewv)
            pltpu.async_copy(h_hbm.at[srcv], rows, sem).wait()

            def mul(g, carry):
                wv = ewv[pl.ds(g * 16, 16)]
                for l in range(16):
                    w = wv[l]
                    for t in range(F // 16):
                        b = g * 16 + l
                        rows[b, pl.ds(t * 16, 16)] = rows[b, pl.ds(t * 16, 16)] * w
                return carry
            lax.fori_loop(0, CHUNK // 16, mul, 0)

            pltpu.sync_copy(rows, acc.at[dstv], add=True)
            return _

        lax.fori_loop(0, NCHUNK, chunk_body, 0)
        plsc.subcore_barrier()

        r = s * ROWS_PER_SUB
        pltpu.sync_copy(acc.at[pl.ds(r, ROWS_PER_SUB)],
                        out_hbm.at[c, pl.ds(r, ROWS_PER_SUB)])

    return sc_agg


def _make_sc_deg():
    mesh = plsc.VectorSubcoreMesh(core_axis_name="c", subcore_axis_name="s")

    @functools.partial(
        pl.kernel,
        out_type=jax.ShapeDtypeStruct((2, NPAD), F32),
        mesh=mesh,
        compiler_params=pltpu.CompilerParams(use_tc_tiling_on_sc=False),
        scratch_types=[
            pltpu.VMEM((CHUNK,), jnp.int32),
            pltpu.VMEM((CHUNK,), F32),
            pltpu.VMEM_SHARED((NPAD,), F32),
        ],
    )
    def sc_deg(dst_hbm, ew_hbm, out_hbm, dstv, ewv, acc):
        c = lax.axis_index("c")
        s = lax.axis_index("s")

        def zb(i, _):
            ewv[pl.ds(i * 16, 16)] = jnp.zeros((16,), F32)
            return _
        lax.fori_loop(0, CHUNK // 16, zb, 0)
        _zero_acc_slice(ewv, acc, s)
        plsc.subcore_barrier()

        base = (c * 16 + s) * EDGES_PER_TILE

        def chunk_body(i, _):
            e0 = base + i * CHUNK
            pltpu.sync_copy(dst_hbm.at[pl.ds(e0, CHUNK)], dstv)
            pltpu.sync_copy(ew_hbm.at[pl.ds(e0, CHUNK)], ewv)
            pltpu.sync_copy(ewv, acc.at[dstv], add=True)
            return _

        lax.fori_loop(0, NCHUNK, chunk_body, 0)
        plsc.subcore_barrier()

        r = s * ROWS_PER_SUB
        pltpu.sync_copy(acc.at[pl.ds(r, ROWS_PER_SUB)],
                        out_hbm.at[c, pl.ds(r, ROWS_PER_SUB)])

    return sc_deg


def _dinv(degp_blk):
    return lax.rsqrt(1.0 + degp_blk[0, :] + degp_blk[1, :])


def _tc_a(x, degp, W1):
    def body(x_ref, degp_ref, w_ref, out_ref):
        dinv = _dinv(degp_ref[...])
        t1 = jnp.dot(x_ref[...], w_ref[...], preferred_element_type=F32,
                     precision=lax.Precision.HIGHEST)
        out_ref[...] = t1 * dinv[:, None]

    return pl.pallas_call(
        body,
        grid=(GRID,),
        in_specs=[
            pl.BlockSpec((BLK, 128), lambda i: (i, 0)),
            pl.BlockSpec((2, BLK), lambda i: (0, i)),
            pl.BlockSpec((128, 16), lambda i: (0, 0)),
        ],
        out_specs=pl.BlockSpec((BLK, 16), lambda i: (i, 0)),
        out_shape=jax.ShapeDtypeStruct((NPAD, 16), F32),
    )(x, degp, W1)


def _tc_b(P1, g1p, degp, b1):
    def body(p_ref, g_ref, degp_ref, b_ref, out_ref):
        dinv = _dinv(degp_ref[...])[:, None]
        agg = dinv * (p_ref[0] + p_ref[1] + g_ref[...])
        out_ref[...] = dinv * jax.nn.relu(agg + b_ref[...])

    return pl.pallas_call(
        body,
        grid=(GRID,),
        in_specs=[
            pl.BlockSpec((2, BLK, 16), lambda i: (0, i, 0)),
            pl.BlockSpec((BLK, 16), lambda i: (i, 0)),
            pl.BlockSpec((2, BLK), lambda i: (0, i)),
            pl.BlockSpec((1, 16), lambda i: (0, 0)),
        ],
        out_specs=pl.BlockSpec((BLK, 16), lambda i: (i, 0)),
        out_shape=jax.ShapeDtypeStruct((NPAD, 16), F32),
    )(P1, g1p, degp, b1.reshape(1, 16))


def _tc_c(P2, h1pp, degp, W2, b2):
    def body(p_ref, g_ref, degp_ref, w_ref, b_ref, out_ref):
        dinv = _dinv(degp_ref[...])[:, None]
        agg = dinv * (p_ref[0] + p_ref[1] + g_ref[...])
        h2 = jax.nn.relu(jnp.dot(agg, w_ref[...], preferred_element_type=F32,
                                 precision=lax.Precision.HIGHEST) + b_ref[...])
        out_ref[...] = dinv * h2

    return pl.pallas_call(
        body,
        grid=(GRID,),
        in_specs=[
            pl.BlockSpec((2, BLK, 16), lambda i: (0, i, 0)),
            pl.BlockSpec((BLK, 16), lambda i: (i, 0)),
            pl.BlockSpec((2, BLK), lambda i: (0, i)),
            pl.BlockSpec((16, 32), lambda i: (0, 0)),
            pl.BlockSpec((1, 32), lambda i: (0, 0)),
        ],
        out_specs=pl.BlockSpec((BLK, 32), lambda i: (i, 0)),
        out_shape=jax.ShapeDtypeStruct((NPAD, 32), F32),
    )(P2, h1pp, degp, W2, b2.reshape(1, 32))


def _tc_d(P3, h2p, degp, W3, b3, batch2d, Wf, bf):
    def body(p_ref, g_ref, degp_ref, w_ref, b_ref, batch_ref, wf_ref, bf_ref,
             out_ref, pooled_scr, cnt_scr):
        i = pl.program_id(0)

        @pl.when(i == 0)
        def _():
            pooled_scr[...] = jnp.zeros((G, 64), F32)
            cnt_scr[...] = jnp.zeros((1, G), F32)

        dinv = _dinv(degp_ref[...])[:, None]
        agg = dinv * (p_ref[0] + p_ref[1] + g_ref[...])
        h3 = jax.nn.relu(jnp.dot(agg, w_ref[...], preferred_element_type=F32,
                                 precision=lax.Precision.HIGHEST) + b_ref[...])
        gid = lax.broadcasted_iota(jnp.int32, (G, 1), 0)
        onehot = (batch_ref[...] == gid).astype(F32)
        pooled_scr[...] += jnp.dot(onehot, h3, preferred_element_type=F32,
                                   precision=lax.Precision.HIGHEST)
        cnt_scr[...] += jnp.sum(onehot, axis=1)[None, :]

        @pl.when(i == GRID - 1)
        def _():
            cnt = jnp.maximum(cnt_scr[...], 1.0)
            pooled = pooled_scr[...] / cnt.reshape(G, 1)
            out_ref[...] = jnp.dot(pooled, wf_ref[...], preferred_element_type=F32,
                                   precision=lax.Precision.HIGHEST) + bf_ref[...]

    return pl.pallas_call(
        body,
        grid=(GRID,),
        in_specs=[
            pl.BlockSpec((2, BLK, 32), lambda i: (0, i, 0)),
            pl.BlockSpec((BLK, 32), lambda i: (i, 0)),
            pl.BlockSpec((2, BLK), lambda i: (0, i)),
            pl.BlockSpec((32, 64), lambda i: (0, 0)),
            pl.BlockSpec((1, 64), lambda i: (0, 0)),
            pl.BlockSpec((1, BLK), lambda i: (0, i)),
            pl.BlockSpec((64, 10), lambda i: (0, 0)),
            pl.BlockSpec((1, 10), lambda i: (0, 0)),
        ],
        out_specs=pl.BlockSpec((G, 10), lambda i: (0, 0)),
        out_shape=jax.ShapeDtypeStruct((G, 10), F32),
        scratch_shapes=[
            pltpu.VMEM((G, 64), F32),
            pltpu.VMEM((1, G), F32),
        ],
    )(P3, h2p, degp, W3, b3.reshape(1, 64), batch2d, Wf, bf.reshape(1, 10))


_sc_agg16 = _make_sc_agg(16)
_sc_agg32 = _make_sc_agg(32)
_sc_deg = _make_sc_deg()


def kernel(x, edge_index, edge_attr, edge_weights, batch,
           W1, b1, W2, b2, W3, b3, Wf, bf):
    del edge_attr
    src = edge_index[0]
    dst = edge_index[1]
    pad = EPAD - E
    srcp = jnp.concatenate([src, jnp.zeros((pad,), jnp.int32)])
    dstp = jnp.concatenate([dst, jnp.zeros((pad,), jnp.int32)])
    ewp = jnp.concatenate([edge_weights, jnp.zeros((pad,), F32)])
    npad = NPAD - N
    xp = jnp.concatenate([x, jnp.zeros((npad, 128), F32)])
    batch2d = jnp.concatenate([batch, jnp.full((npad,), G, jnp.int32)]).reshape(1, NPAD)

    degp = _sc_deg(dstp, ewp)
    g1p = _tc_a(xp, degp, W1)
    P1 = _sc_agg16(g1p, srcp, dstp, ewp)
    h1pp = _tc_b(P1, g1p, degp, b1)
    P2 = _sc_agg16(h1pp, srcp, dstp, ewp)
    h2p = _tc_c(P2, h1pp, degp, W2, b2)
    P3 = _sc_agg32(h2p, srcp, dstp, ewp)
    return _tc_d(P3, h2p, degp, W3, b3, batch2d, Wf, bf)


# speedup vs baseline: 18.2137x; 1.4518x over previous
import functools

import jax
import jax.numpy as jnp
from jax import lax
from jax.experimental import pallas as pl
from jax.experimental.pallas import tpu as pltpu
from jax.experimental.pallas import tpu_sc as plsc

N = 50000
E = 800000
NPAD = 51200
EPAD = 819200
ROWS_PER_SUB = 3200
EDGES_PER_TILE = EPAD // 32
CHUNK = 128
NCHUNK = EDGES_PER_TILE // CHUNK
BLK = 2048
GRID = NPAD // BLK
G = 64
F32 = jnp.float32


def _zero_acc_slice(zbuf, acc, s):
    def z(k, _):
        pltpu.sync_copy(zbuf, acc.at[pl.ds(s * ROWS_PER_SUB + k * 128, 128)])
        return _
    lax.fori_loop(0, 25, z, 0)


def _make_sc_agg(F):
    mesh = plsc.VectorSubcoreMesh(core_axis_name="c", subcore_axis_name="s")
    NROW = EDGES_PER_TILE // CHUNK

    @functools.partial(
        pl.kernel,
        out_type=jax.ShapeDtypeStruct((2, NPAD, F), F32),
        mesh=mesh,
        compiler_params=pltpu.CompilerParams(use_tc_tiling_on_sc=False),
        scratch_types=[
            pltpu.VMEM((NROW, 128), jnp.int32),
            pltpu.VMEM((NROW, 128), jnp.int32),
            pltpu.VMEM((NROW, 128), F32),
            pltpu.VMEM((CHUNK, F), F32),
            pltpu.VMEM((CHUNK, F), F32),
            pltpu.VMEM((CHUNK, F), F32),
            pltpu.VMEM((CHUNK, F), F32),
            pltpu.VMEM_SHARED((NPAD, F), F32),
            pltpu.SemaphoreType.DMA,
        ],
    )
    def sc_agg(h_hbm, src_hbm, dst_hbm, ew_hbm, out_hbm,
               src_all, dst_all, ew_all, r0, r1, r2, r3, acc, g0):
        c = lax.axis_index("c")
        s = lax.axis_index("s")

        def zb(i, _):
            for t in range(F // 16):
                r0[i, pl.ds(t * 16, 16)] = jnp.zeros((16,), F32)
            return _
        lax.fori_loop(0, CHUNK, zb, 0)
        _zero_acc_slice(r0, acc, s)

        rowbase = (c * 16 + s) * NROW
        pltpu.sync_copy(src_hbm.at[pl.ds(rowbase, NROW)], src_all)
        pltpu.sync_copy(dst_hbm.at[pl.ds(rowbase, NROW)], dst_all)
        pltpu.sync_copy(ew_hbm.at[pl.ds(rowbase, NROW)], ew_all)
        plsc.subcore_barrier()

        def outer(i, car):
            pltpu.async_copy(h_hbm.at[src_all.at[i]], r0, g0).wait()

            def mul(q, carry):
                wv = ew_all[i, pl.ds(q * 16, 16)]
                for l in range(16):
                    w = wv[l]
                    for t in range(F // 16):
                        e = q * 16 + l
                        r0[e, pl.ds(t * 16, 16)] = r0[e, pl.ds(t * 16, 16)] * w
                return carry
            lax.fori_loop(0, CHUNK // 16, mul, 0)

            pltpu.sync_copy(r0, acc.at[dst_all.at[i]], add=True)
            return car

        lax.fori_loop(0, NROW, outer, 0)

        plsc.subcore_barrier()

        def wb(k, car):
            r = s * ROWS_PER_SUB + k * 128
            pltpu.sync_copy(acc.at[pl.ds(r, 128)], r0)
            pltpu.sync_copy(r0, out_hbm.at[c, pl.ds(r, 128)])
            return car
        lax.fori_loop(0, ROWS_PER_SUB // 128, wb, 0)

    return sc_agg


def _make_sc_deg():
    mesh = plsc.VectorSubcoreMesh(core_axis_name="c", subcore_axis_name="s")
    NROW = EDGES_PER_TILE // CHUNK

    @functools.partial(
        pl.kernel,
        out_type=jax.ShapeDtypeStruct((2, NPAD), F32),
        mesh=mesh,
        compiler_params=pltpu.CompilerParams(use_tc_tiling_on_sc=False),
        scratch_types=[
            pltpu.VMEM((NROW, 128), jnp.int32),
            pltpu.VMEM((NROW, 128), F32),
            pltpu.VMEM((128,), F32),
            pltpu.VMEM_SHARED((NPAD,), F32),
            pltpu.SemaphoreType.DMA,
            pltpu.SemaphoreType.DMA,
            pltpu.SemaphoreType.DMA,
            pltpu.SemaphoreType.DMA,
        ],
    )
    def sc_deg(dst_hbm, ew_hbm, out_hbm, dst_all, ew_all, zbuf, acc,
               s0, s1, s2, s3):
        c = lax.axis_index("c")
        s = lax.axis_index("s")
        ssem = (s0, s1, s2, s3)

        def zb(i, _):
            zbuf[pl.ds(i * 16, 16)] = jnp.zeros((16,), F32)
            return _
        lax.fori_loop(0, 128 // 16, zb, 0)
        _zero_acc_slice(zbuf, acc, s)

        rowbase = (c * 16 + s) * NROW
        pltpu.sync_copy(dst_hbm.at[pl.ds(rowbase, NROW)], dst_all)
        pltpu.sync_copy(ew_hbm.at[pl.ds(rowbase, NROW)], ew_all)
        plsc.subcore_barrier()

        def outer(g, car):
            for b in range(4):
                i = g * 4 + b

                @pl.when(i >= 4)
                def _():
                    pltpu.make_async_copy(
                        ew_all.at[i - 4], acc.at[dst_all.at[i - 4]], ssem[b]).wait()

                pltpu.async_copy(ew_all.at[i], acc.at[dst_all.at[i]], ssem[b], add=True)
            return car

        lax.fori_loop(0, NROW // 4, outer, 0)
        for b in range(4):
            i = NROW - 4 + b
            pltpu.make_async_copy(ew_all.at[i], acc.at[dst_all.at[i]], ssem[b]).wait()
        plsc.subcore_barrier()

        def wb(k, car):
            r = s * ROWS_PER_SUB + k * 128
            pltpu.sync_copy(acc.at[pl.ds(r, 128)], zbuf)
            pltpu.sync_copy(zbuf, out_hbm.at[c, pl.ds(r, 128)])
            return car
        lax.fori_loop(0, ROWS_PER_SUB // 128, wb, 0)

    return sc_deg


def _dinv(degp_blk):
    return lax.rsqrt(1.0 + degp_blk[0, :] + degp_blk[1, :])


def _tc_a(x, degp, W1):
    def body(x_ref, degp_ref, w_ref, out_ref):
        dinv = _dinv(degp_ref[...])
        t1 = jnp.dot(x_ref[...], w_ref[...], preferred_element_type=F32,
                     precision=lax.Precision.HIGHEST)
        out_ref[...] = t1 * dinv[:, None]

    return pl.pallas_call(
        body,
        grid=(GRID,),
        in_specs=[
            pl.BlockSpec((BLK, 128), lambda i: (i, 0)),
            pl.BlockSpec((2, BLK), lambda i: (0, i)),
            pl.BlockSpec((128, 16), lambda i: (0, 0)),
        ],
        out_specs=pl.BlockSpec((BLK, 16), lambda i: (i, 0)),
        out_shape=jax.ShapeDtypeStruct((NPAD, 16), F32),
    )(x, degp, W1)


def _tc_b(P1, g1p, degp, b1):
    def body(p_ref, g_ref, degp_ref, b_ref, out_ref):
        dinv = _dinv(degp_ref[...])[:, None]
        agg = dinv * (p_ref[0] + p_ref[1] + g_ref[...])
        out_ref[...] = dinv * jax.nn.relu(agg + b_ref[...])

    return pl.pallas_call(
        body,
        grid=(GRID,),
        in_specs=[
            pl.BlockSpec((2, BLK, 16), lambda i: (0, i, 0)),
            pl.BlockSpec((BLK, 16), lambda i: (i, 0)),
            pl.BlockSpec((2, BLK), lambda i: (0, i)),
            pl.BlockSpec((1, 16), lambda i: (0, 0)),
        ],
        out_specs=pl.BlockSpec((BLK, 16), lambda i: (i, 0)),
        out_shape=jax.ShapeDtypeStruct((NPAD, 16), F32),
    )(P1, g1p, degp, b1.reshape(1, 16))


def _tc_c(P2, h1pp, degp, W2, b2):
    def body(p_ref, g_ref, degp_ref, w_ref, b_ref, out_ref):
        dinv = _dinv(degp_ref[...])[:, None]
        agg = dinv * (p_ref[0] + p_ref[1] + g_ref[...])
        h2 = jax.nn.relu(jnp.dot(agg, w_ref[...], preferred_element_type=F32,
                                 precision=lax.Precision.HIGHEST) + b_ref[...])
        out_ref[...] = dinv * h2

    return pl.pallas_call(
        body,
        grid=(GRID,),
        in_specs=[
            pl.BlockSpec((2, BLK, 16), lambda i: (0, i, 0)),
            pl.BlockSpec((BLK, 16), lambda i: (i, 0)),
            pl.BlockSpec((2, BLK), lambda i: (0, i)),
            pl.BlockSpec((16, 32), lambda i: (0, 0)),
            pl.BlockSpec((1, 32), lambda i: (0, 0)),
        ],
        out_specs=pl.BlockSpec((BLK, 32), lambda i: (i, 0)),
        out_shape=jax.ShapeDtypeStruct((NPAD, 32), F32),
    )(P2, h1pp, degp, W2, b2.reshape(1, 32))


def _tc_d(P3a, P3b, h2p, degp, W3, b3, batch2d, Wf, bf):
    def body(pa_ref, pb_ref, g_ref, degp_ref, w_ref, b_ref, batch_ref,
             wf_ref, bf_ref, out_ref, pooled_scr, cnt_scr):
        i = pl.program_id(0)

        @pl.when(i == 0)
        def _():
            pooled_scr[...] = jnp.zeros((G, 64), F32)
            cnt_scr[...] = jnp.zeros((1, G), F32)

        dinv = _dinv(degp_ref[...])[:, None]
        p3 = jnp.concatenate([pa_ref[0] + pa_ref[1], pb_ref[0] + pb_ref[1]], axis=1)
        agg = dinv * (p3 + g_ref[...])
        h3 = jax.nn.relu(jnp.dot(agg, w_ref[...], preferred_element_type=F32,
                                 precision=lax.Precision.HIGHEST) + b_ref[...])
        gid = lax.broadcasted_iota(jnp.int32, (G, 1), 0)
        onehot = (batch_ref[...] == gid).astype(F32)
        pooled_scr[...] += jnp.dot(onehot, h3, preferred_element_type=F32,
                                   precision=lax.Precision.HIGHEST)
        cnt_scr[...] += jnp.sum(onehot, axis=1)[None, :]

        @pl.when(i == GRID - 1)
        def _():
            cnt = jnp.maximum(cnt_scr[...], 1.0)
            pooled = pooled_scr[...] / cnt.reshape(G, 1)
            out_ref[...] = jnp.dot(pooled, wf_ref[...], preferred_element_type=F32,
                                   precision=lax.Precision.HIGHEST) + bf_ref[...]

    return pl.pallas_call(
        body,
        grid=(GRID,),
        in_specs=[
            pl.BlockSpec((2, BLK, 16), lambda i: (0, i, 0)),
            pl.BlockSpec((2, BLK, 16), lambda i: (0, i, 0)),
            pl.BlockSpec((BLK, 32), lambda i: (i, 0)),
            pl.BlockSpec((2, BLK), lambda i: (0, i)),
            pl.BlockSpec((32, 64), lambda i: (0, 0)),
            pl.BlockSpec((1, 64), lambda i: (0, 0)),
            pl.BlockSpec((1, BLK), lambda i: (0, i)),
            pl.BlockSpec((64, 10), lambda i: (0, 0)),
            pl.BlockSpec((1, 10), lambda i: (0, 0)),
        ],
        out_specs=pl.BlockSpec((G, 10), lambda i: (0, 0)),
        out_shape=jax.ShapeDtypeStruct((G, 10), F32),
        scratch_shapes=[
            pltpu.VMEM((G, 64), F32),
            pltpu.VMEM((1, G), F32),
        ],
    )(P3a, P3b, h2p, degp, W3, b3.reshape(1, 64), batch2d, Wf, bf.reshape(1, 10))


_sc_agg16 = _make_sc_agg(16)
_sc_deg = _make_sc_deg()


def kernel(x, edge_index, edge_attr, edge_weights, batch,
           W1, b1, W2, b2, W3, b3, Wf, bf):
    del edge_attr
    src = edge_index[0]
    dst = edge_index[1]
    pad = EPAD - E
    srcp = jnp.concatenate([src, jnp.zeros((pad,), jnp.int32)]).reshape(-1, 128)
    dstp = jnp.concatenate([dst, jnp.zeros((pad,), jnp.int32)]).reshape(-1, 128)
    ewp = jnp.concatenate([edge_weights, jnp.zeros((pad,), F32)]).reshape(-1, 128)
    npad = NPAD - N
    xp = jnp.concatenate([x, jnp.zeros((npad, 128), F32)])
    batch2d = jnp.concatenate([batch, jnp.full((npad,), G, jnp.int32)]).reshape(1, NPAD)

    degp = _sc_deg(dstp, ewp)
    g1p = _tc_a(xp, degp, W1)
    P1 = _sc_agg16(g1p, srcp, dstp, ewp)
    h1pp = _tc_b(P1, g1p, degp, b1)
    P2 = _sc_agg16(h1pp, srcp, dstp, ewp)
    h2p = _tc_c(P2, h1pp, degp, W2, b2)
    P3a = _sc_agg16(h2p[:, :16], srcp, dstp, ewp)
    P3b = _sc_agg16(h2p[:, 16:], srcp, dstp, ewp)
    return _tc_d(P3a, P3b, h2p, degp, W3, b3, batch2d, Wf, bf)


# speedup vs baseline: 18.2237x; 1.0006x over previous
import functools

import jax
import jax.numpy as jnp
from jax import lax
from jax.experimental import pallas as pl
from jax.experimental.pallas import tpu as pltpu
from jax.experimental.pallas import tpu_sc as plsc

N = 50000
E = 800000
NPAD = 51200
EPAD = 819200
ROWS_PER_SUB = 3200
EDGES_PER_TILE = EPAD // 32
CHUNK = 128
NCHUNK = EDGES_PER_TILE // CHUNK
BLK = 2048
GRID = NPAD // BLK
G = 64
F32 = jnp.float32


def _zero_acc_slice(zbuf, acc, s):
    def z(k, _):
        pltpu.sync_copy(zbuf, acc.at[pl.ds(s * ROWS_PER_SUB + k * 128, 128)])
        return _
    lax.fori_loop(0, 25, z, 0)


def _make_sc_agg(F):
    mesh = plsc.VectorSubcoreMesh(core_axis_name="c", subcore_axis_name="s")
    NROW = EDGES_PER_TILE // CHUNK

    @functools.partial(
        pl.kernel,
        out_type=jax.ShapeDtypeStruct((2, NPAD, F), F32),
        mesh=mesh,
        compiler_params=pltpu.CompilerParams(use_tc_tiling_on_sc=False),
        scratch_types=[
            pltpu.VMEM((NROW, 128), jnp.int32),
            pltpu.VMEM((NROW, 128), jnp.int32),
            pltpu.VMEM((NROW, 128), F32),
            pltpu.VMEM((CHUNK, F), F32),
            pltpu.VMEM_SHARED((NPAD, F), F32),
            pltpu.SemaphoreType.DMA,
        ],
    )
    def sc_agg(h_hbm, src_hbm, dst_hbm, ew_hbm, out_hbm,
               src_all, dst_all, ew_all, r0, acc, g0):
        c = lax.axis_index("c")
        s = lax.axis_index("s")

        def zb(i, _):
            for t in range(F // 16):
                r0[i, pl.ds(t * 16, 16)] = jnp.zeros((16,), F32)
            return _
        lax.fori_loop(0, CHUNK, zb, 0)
        _zero_acc_slice(r0, acc, s)

        rowbase = (c * 16 + s) * NROW
        pltpu.sync_copy(src_hbm.at[pl.ds(rowbase, NROW)], src_all)
        pltpu.sync_copy(dst_hbm.at[pl.ds(rowbase, NROW)], dst_all)
        pltpu.sync_copy(ew_hbm.at[pl.ds(rowbase, NROW)], ew_all)
        plsc.subcore_barrier()

        def outer(i, car):
            pltpu.async_copy(h_hbm.at[src_all.at[i]], r0, g0).wait()

            def mul(q, carry):
                wv = ew_all[i, pl.ds(q * 16, 16)]
                for l in range(16):
                    w = wv[l]
                    for t in range(F // 16):
                        e = q * 16 + l
                        r0[e, pl.ds(t * 16, 16)] = r0[e, pl.ds(t * 16, 16)] * w
                return carry
            lax.fori_loop(0, CHUNK // 16, mul, 0)

            pltpu.sync_copy(r0, acc.at[dst_all.at[i]], add=True)
            return car

        lax.fori_loop(0, NROW, outer, 0)

        plsc.subcore_barrier()

        def wb(k, car):
            r = s * ROWS_PER_SUB + k * 128
            pltpu.sync_copy(acc.at[pl.ds(r, 128)], r0)
            pltpu.sync_copy(r0, out_hbm.at[c, pl.ds(r, 128)])
            return car
        lax.fori_loop(0, ROWS_PER_SUB // 128, wb, 0)

    return sc_agg


def _make_sc_deg():
    mesh = plsc.VectorSubcoreMesh(core_axis_name="c", subcore_axis_name="s")
    NROW = EDGES_PER_TILE // CHUNK

    @functools.partial(
        pl.kernel,
        out_type=jax.ShapeDtypeStruct((2, NPAD), F32),
        mesh=mesh,
        compiler_params=pltpu.CompilerParams(use_tc_tiling_on_sc=False),
        scratch_types=[
            pltpu.VMEM((NROW, 128), jnp.int32),
            pltpu.VMEM((NROW, 128), F32),
            pltpu.VMEM((128,), F32),
            pltpu.VMEM_SHARED((NPAD,), F32),
            pltpu.SemaphoreType.DMA,
            pltpu.SemaphoreType.DMA,
            pltpu.SemaphoreType.DMA,
            pltpu.SemaphoreType.DMA,
        ],
    )
    def sc_deg(dst_hbm, ew_hbm, out_hbm, dst_all, ew_all, zbuf, acc,
               s0, s1, s2, s3):
        c = lax.axis_index("c")
        s = lax.axis_index("s")
        ssem = (s0, s1, s2, s3)

        def zb(i, _):
            zbuf[pl.ds(i * 16, 16)] = jnp.zeros((16,), F32)
            return _
        lax.fori_loop(0, 128 // 16, zb, 0)
        _zero_acc_slice(zbuf, acc, s)

        rowbase = (c * 16 + s) * NROW
        pltpu.sync_copy(dst_hbm.at[pl.ds(rowbase, NROW)], dst_all)
        pltpu.sync_copy(ew_hbm.at[pl.ds(rowbase, NROW)], ew_all)
        plsc.subcore_barrier()

        def outer(g, car):
            for b in range(4):
                i = g * 4 + b

                @pl.when(i >= 4)
                def _():
                    pltpu.make_async_copy(
                        ew_all.at[i - 4], acc.at[dst_all.at[i - 4]], ssem[b]).wait()

                pltpu.async_copy(ew_all.at[i], acc.at[dst_all.at[i]], ssem[b], add=True)
            return car

        lax.fori_loop(0, NROW // 4, outer, 0)
        for b in range(4):
            i = NROW - 4 + b
            pltpu.make_async_copy(ew_all.at[i], acc.at[dst_all.at[i]], ssem[b]).wait()
        plsc.subcore_barrier()

        def wb(k, car):
            r = s * ROWS_PER_SUB + k * 128
            pltpu.sync_copy(acc.at[pl.ds(r, 128)], zbuf)
            pltpu.sync_copy(zbuf, out_hbm.at[c, pl.ds(r, 128)])
            return car
        lax.fori_loop(0, ROWS_PER_SUB // 128, wb, 0)

    return sc_deg


def _dinv(degp_blk):
    return lax.rsqrt(1.0 + degp_blk[0, :] + degp_blk[1, :])


def _tc_a(x, degp, W1):
    def body(x_ref, degp_ref, w_ref, out_ref):
        dinv = _dinv(degp_ref[...])
        t1 = jnp.dot(x_ref[...], w_ref[...], preferred_element_type=F32,
                     precision=lax.Precision.HIGHEST)
        out_ref[...] = t1 * dinv[:, None]

    return pl.pallas_call(
        body,
        grid=(GRID,),
        in_specs=[
            pl.BlockSpec((BLK, 128), lambda i: (i, 0)),
            pl.BlockSpec((2, BLK), lambda i: (0, i)),
            pl.BlockSpec((128, 16), lambda i: (0, 0)),
        ],
        out_specs=pl.BlockSpec((BLK, 16), lambda i: (i, 0)),
        out_shape=jax.ShapeDtypeStruct((NPAD, 16), F32),
    )(x, degp, W1)


def _tc_b(P1, g1p, degp, b1):
    def body(p_ref, g_ref, degp_ref, b_ref, out_ref):
        dinv = _dinv(degp_ref[...])[:, None]
        agg = dinv * (p_ref[0] + p_ref[1] + g_ref[...])
        out_ref[...] = dinv * jax.nn.relu(agg + b_ref[...])

    return pl.pallas_call(
        body,
        grid=(GRID,),
        in_specs=[
            pl.BlockSpec((2, BLK, 16), lambda i: (0, i, 0)),
            pl.BlockSpec((BLK, 16), lambda i: (i, 0)),
            pl.BlockSpec((2, BLK), lambda i: (0, i)),
            pl.BlockSpec((1, 16), lambda i: (0, 0)),
        ],
        out_specs=pl.BlockSpec((BLK, 16), lambda i: (i, 0)),
        out_shape=jax.ShapeDtypeStruct((NPAD, 16), F32),
    )(P1, g1p, degp, b1.reshape(1, 16))


def _tc_c(P2, h1pp, degp, W2, b2):
    def body(p_ref, g_ref, degp_ref, w_ref, b_ref, out_ref):
        dinv = _dinv(degp_ref[...])[:, None]
        agg = dinv * (p_ref[0] + p_ref[1] + g_ref[...])
        h2 = jax.nn.relu(jnp.dot(agg, w_ref[...], preferred_element_type=F32,
                                 precision=lax.Precision.HIGHEST) + b_ref[...])
        out_ref[...] = dinv * h2

    return pl.pallas_call(
        body,
        grid=(GRID,),
        in_specs=[
            pl.BlockSpec((2, BLK, 16), lambda i: (0, i, 0)),
            pl.BlockSpec((BLK, 16), lambda i: (i, 0)),
            pl.BlockSpec((2, BLK), lambda i: (0, i)),
            pl.BlockSpec((16, 32), lambda i: (0, 0)),
            pl.BlockSpec((1, 32), lambda i: (0, 0)),
        ],
        out_specs=pl.BlockSpec((BLK, 32), lambda i: (i, 0)),
        out_shape=jax.ShapeDtypeStruct((NPAD, 32), F32),
    )(P2, h1pp, degp, W2, b2.reshape(1, 32))


def _tc_d(P3a, P3b, h2p, degp, W3, b3, batch2d, Wf, bf):
    def body(pa_ref, pb_ref, g_ref, degp_ref, w_ref, b_ref, batch_ref,
             wf_ref, bf_ref, out_ref, pooled_scr, cnt_scr):
        i = pl.program_id(0)

        @pl.when(i == 0)
        def _():
            pooled_scr[...] = jnp.zeros((G, 64), F32)
            cnt_scr[...] = jnp.zeros((1, G), F32)

        dinv = _dinv(degp_ref[...])[:, None]
        p3 = jnp.concatenate([pa_ref[0] + pa_ref[1], pb_ref[0] + pb_ref[1]], axis=1)
        agg = dinv * (p3 + g_ref[...])
        h3 = jax.nn.relu(jnp.dot(agg, w_ref[...], preferred_element_type=F32,
                                 precision=lax.Precision.HIGHEST) + b_ref[...])
        gid = lax.broadcasted_iota(jnp.int32, (G, 1), 0)
        onehot = (batch_ref[...] == gid).astype(F32)
        pooled_scr[...] += jnp.dot(onehot, h3, preferred_element_type=F32,
                                   precision=lax.Precision.HIGHEST)
        cnt_scr[...] += jnp.sum(onehot, axis=1)[None, :]

        @pl.when(i == GRID - 1)
        def _():
            cnt = jnp.maximum(cnt_scr[...], 1.0)
            pooled = pooled_scr[...] / cnt.reshape(G, 1)
            out_ref[...] = jnp.dot(pooled, wf_ref[...], preferred_element_type=F32,
                                   precision=lax.Precision.HIGHEST) + bf_ref[...]

    return pl.pallas_call(
        body,
        grid=(GRID,),
        in_specs=[
            pl.BlockSpec((2, BLK, 16), lambda i: (0, i, 0)),
            pl.BlockSpec((2, BLK, 16), lambda i: (0, i, 0)),
            pl.BlockSpec((BLK, 32), lambda i: (i, 0)),
            pl.BlockSpec((2, BLK), lambda i: (0, i)),
            pl.BlockSpec((32, 64), lambda i: (0, 0)),
            pl.BlockSpec((1, 64), lambda i: (0, 0)),
            pl.BlockSpec((1, BLK), lambda i: (0, i)),
            pl.BlockSpec((64, 10), lambda i: (0, 0)),
            pl.BlockSpec((1, 10), lambda i: (0, 0)),
        ],
        out_specs=pl.BlockSpec((G, 10), lambda i: (0, 0)),
        out_shape=jax.ShapeDtypeStruct((G, 10), F32),
        scratch_shapes=[
            pltpu.VMEM((G, 64), F32),
            pltpu.VMEM((1, G), F32),
        ],
    )(P3a, P3b, h2p, degp, W3, b3.reshape(1, 64), batch2d, Wf, bf.reshape(1, 10))


_sc_agg16 = _make_sc_agg(16)
_sc_deg = _make_sc_deg()


def kernel(x, edge_index, edge_attr, edge_weights, batch,
           W1, b1, W2, b2, W3, b3, Wf, bf):
    del edge_attr
    src = edge_index[0]
    dst = edge_index[1]
    pad = EPAD - E
    srcp = jnp.concatenate([src, jnp.zeros((pad,), jnp.int32)]).reshape(-1, 128)
    dstp = jnp.concatenate([dst, jnp.zeros((pad,), jnp.int32)]).reshape(-1, 128)
    ewp = jnp.concatenate([edge_weights, jnp.zeros((pad,), F32)]).reshape(-1, 128)
    npad = NPAD - N
    xp = jnp.concatenate([x, jnp.zeros((npad, 128), F32)])
    batch2d = jnp.concatenate([batch, jnp.full((npad,), G, jnp.int32)]).reshape(1, NPAD)

    degp = _sc_deg(dstp, ewp)
    g1p = _tc_a(xp, degp, W1)
    P1 = _sc_agg16(g1p, srcp, dstp, ewp)
    h1pp = _tc_b(P1, g1p, degp, b1)
    P2 = _sc_agg16(h1pp, srcp, dstp, ewp)
    h2p = _tc_c(P2, h1pp, degp, W2, b2)
    P3a = _sc_agg16(h2p[:, :16], srcp, dstp, ewp)
    P3b = _sc_agg16(h2p[:, 16:], srcp, dstp, ewp)
    return _tc_d(P3a, P3b, h2p, degp, W3, b3, batch2d, Wf, bf)
